# baseline (speedup 1.0000x reference)
import jax
import jax.numpy as jnp
from jax.experimental import pallas as pl

BATCH = 16384
E = 64
BT = 512


def _body(x_ref, c_ref, r_ref, out_ref, idx_ref):
    x = x_ref[...]
    c = c_ref[...]
    r = r_ref[...]

    cn2 = jnp.sum(c * c, axis=1)
    xn2 = jnp.sum(x * x, axis=1)
    g = jax.lax.dot_general(x, c, (((1,), (1,)), ((), ())),
                            preferred_element_type=jnp.float32)
    d2m = xn2[:, None] + (cn2[None, :] - 2.0 * g)

    lane = jax.lax.broadcasted_iota(jnp.int32, (BT, E), 1)
    a1 = jnp.argmin(d2m, axis=1)
    masked = jnp.where(lane == a1[:, None], jnp.inf, d2m)
    a2 = jnp.argmin(masked, axis=1)

    h1 = (lane == a1[:, None]).astype(jnp.float32)
    h2 = (lane == a2[:, None]).astype(jnp.float32)
    c1 = jnp.dot(h1, c, preferred_element_type=jnp.float32)
    c2 = jnp.dot(h2, c, preferred_element_type=jnp.float32)

    dx1 = x - c1
    dx2 = x - c2
    e1 = jnp.sum(dx1 * dx1, axis=1)
    e2 = jnp.sum(dx2 * dx2, axis=1)

    pred = (e2 < e1) | ((e2 == e1) & (a2 < a1))
    best = jnp.where(pred, a2, a1)
    mind = jnp.sqrt(jnp.where(pred, e2, e1))
    csel = jnp.where(pred[:, None], c2, c1)
    rsel = jnp.where(pred, jnp.sum(h2 * r, axis=1), jnp.sum(h1 * r, axis=1))

    s = 0.1 * jnp.exp(-mind / (rsel + 1e-8))
    out_ref[...] = x * (1.0 - s)[:, None] + csel * s[:, None]
    idx_ref[...] = best[:, None].astype(jnp.int32)


def kernel(expert_activations, attractor_centers, attraction_radii):
    radii2d = attraction_radii.reshape(1, E)
    attracted, closest = pl.pallas_call(
        _body,
        grid=(BATCH // BT,),
        in_specs=[
            pl.BlockSpec((BT, E), lambda i: (i, 0)),
            pl.BlockSpec((E, E), lambda i: (0, 0)),
            pl.BlockSpec((1, E), lambda i: (0, 0)),
        ],
        out_specs=[
            pl.BlockSpec((BT, E), lambda i: (i, 0)),
            pl.BlockSpec((BT, 1), lambda i: (i, 0)),
        ],
        out_shape=[
            jax.ShapeDtypeStruct((BATCH, E), jnp.float32),
            jax.ShapeDtypeStruct((BATCH, 1), jnp.int32),
        ],
    )(expert_activations, attractor_centers, radii2d)
    return attracted, closest.reshape(BATCH)
